# baseline (speedup 1.0000x reference)
import jax
import jax.numpy as jnp
from jax import lax
from jax.experimental import pallas as pl
from jax.experimental.pallas import tpu as pltpu

E = 16
D = 768
H = 3072
N = 32
NSL1 = 8
NSL2 = 6
HS1 = H // NSL1
DS = D // NSL2
JPE = NSL1 + NSL2
T = E * JPE
NB = 10
B1 = NB + 1
B2 = NB + 1


def _w1_copy(w1_hbm, ring, sems, q, r):
    i = q * NSL1 + r
    return pltpu.make_async_copy(
        w1_hbm.at[pl.ds(i * HS1, HS1), :], ring.at[i % B1], sems.at[i % B1])


def _w2_copy(w2_hbm, ring, sems, q, r2):
    i = q * NSL2 + r2
    return pltpu.make_async_copy(
        w2_hbm.at[pl.ds(i * DS, DS), :], ring.at[i % B2], sems.at[i % B2])


def _moe_kernel(x_ref, wg_ref, bg_ref, b1_ref, b2_ref, w1_hbm, w2_hbm,
                out_ref, w1ring, w2ring, hs_ref, sem1, sem2):
    logits = lax.dot_general(
        x_ref[...], wg_ref[...], (((1,), (1,)), ((), ())),
        preferred_element_type=jnp.float32) + bg_ref[...]
    col = lax.broadcasted_iota(jnp.int32, (N, E), 1)
    m1 = jnp.max(logits, axis=-1, keepdims=True)
    i1 = jnp.min(jnp.where(logits == m1, col, E), axis=-1, keepdims=True)
    masked = jnp.where(col == i1, -jnp.inf, logits)
    m2 = jnp.max(masked, axis=-1, keepdims=True)
    i2 = jnp.min(jnp.where(masked == m2, col, E), axis=-1, keepdims=True)
    wa = 1.0 / (1.0 + jnp.exp(m2 - m1))
    wb = 1.0 - wa

    xb = x_ref[...].astype(jnp.bfloat16)

    for t in range(NB):
        q, r = divmod(t, JPE)
        if r < NSL1:
            _w1_copy(w1_hbm, w1ring, sem1, q, r).start()
        else:
            _w2_copy(w2_hbm, w2ring, sem2, q, r - NSL1).start()

    def body(t, carry):
        q = t // JPE
        r = t - q * JPE
        scol = (wa * (i1 == q).astype(jnp.float32)
                + wb * (i2 == q).astype(jnp.float32))

        tn = t + NB
        qn = tn // JPE
        rn = tn - qn * JPE

        @pl.when(jnp.logical_and(tn < T, rn < NSL1))
        def _issue1():
            _w1_copy(w1_hbm, w1ring, sem1, qn, rn).start()

        @pl.when(jnp.logical_and(tn < T, rn >= NSL1))
        def _issue2():
            _w2_copy(w2_hbm, w2ring, sem2, qn, rn - NSL1).start()

        @pl.when(r < NSL1)
        def _up_proj():
            i = q * NSL1 + r
            slot = i % B1
            _w1_copy(w1_hbm, w1ring, sem1, q, r).wait()
            h = lax.dot_general(
                xb, w1ring[slot].astype(jnp.bfloat16), (((1,), (1,)), ((), ())),
                preferred_element_type=jnp.float32)
            h = h + b1_ref[i]
            h = jnp.maximum(h, 0.0) * scol
            hs_ref[r] = h.astype(jnp.bfloat16)

        @pl.when(r >= NSL1)
        def _down_proj():
            r2 = r - NSL1
            i = q * NSL2 + r2
            slot = i % B2
            _w2_copy(w2_hbm, w2ring, sem2, q, r2).wait()
            w2s = w2ring[slot].astype(jnp.bfloat16)
            pb = scol * b2_ref[i]
            for k in range(NSL1):
                pb = pb + lax.dot_general(
                    hs_ref[k], w2s[:, k * HS1:(k + 1) * HS1],
                    (((1,), (1,)), ((), ())),
                    preferred_element_type=jnp.float32)
            prev = jnp.where(q == 0, jnp.zeros((N, DS), jnp.float32),
                             out_ref[r2])
            out_ref[r2] = prev + pb

        return carry

    lax.fori_loop(0, T, body, 0)


@jax.jit
def _moe(x, Wg, bg2, W1, b1, W2, b2):
    return pl.pallas_call(
        _moe_kernel,
        in_specs=[
            pl.BlockSpec(memory_space=pltpu.VMEM),
            pl.BlockSpec(memory_space=pltpu.VMEM),
            pl.BlockSpec(memory_space=pltpu.VMEM),
            pl.BlockSpec(memory_space=pltpu.VMEM),
            pl.BlockSpec(memory_space=pltpu.VMEM),
            pl.BlockSpec(memory_space=pl.ANY),
            pl.BlockSpec(memory_space=pl.ANY),
        ],
        out_specs=pl.BlockSpec(memory_space=pltpu.VMEM),
        out_shape=jax.ShapeDtypeStruct((NSL2, N, DS), jnp.float32),
        scratch_shapes=[
            pltpu.VMEM((B1, HS1, D), jnp.float32),
            pltpu.VMEM((B2, DS, H), jnp.float32),
            pltpu.VMEM((NSL1, N, HS1), jnp.bfloat16),
            pltpu.SemaphoreType.DMA((B1,)),
            pltpu.SemaphoreType.DMA((B2,)),
        ],
    )(x, Wg, bg2, b1, b2, W1, W2)


def kernel(x, Wg, bg, W1, b1, W2, b2):
    out = _moe(x, Wg, bg.reshape(1, E),
               W1.reshape(E * H, D), b1.reshape(E * NSL1, 1, HS1),
               W2.reshape(E * D, H), b2.reshape(E * NSL2, 1, DS))
    return out.transpose(1, 0, 2).reshape(N, D)
